# baseline (device time: 12950 ns/iter reference)
import jax
import jax.numpy as jnp
from jax import lax
from jax.experimental import pallas as pl
from jax.experimental.pallas import tpu as pltpu

N_DEV = 4


def kernel(x):
    m, n = x.shape

    CM = 512
    n_chunks = m // CM

    def body(x_ref, out_ref, comm_ref, buf0_ref, buf1_ref, copy_sems,
             send_sems, recv_sems):
        bufs = (buf0_ref, buf1_ref)
        my = lax.axis_index("i")

        barrier = pltpu.get_barrier_semaphore()
        for k in range(1, N_DEV):
            pl.semaphore_signal(
                barrier,
                inc=1,
                device_id=((my + k) % N_DEV,),
                device_id_type=pl.DeviceIdType.MESH,
            )

        def start_copy(c):
            cp = pltpu.make_async_copy(
                x_ref.at[pl.ds(c * CM, CM), :],
                bufs[c % 2],
                copy_sems.at[c % 2],
            )
            cp.start()
            return cp

        copies = {0: start_copy(0)}
        acc = jnp.zeros((1, n), jnp.float32)
        for c in range(n_chunks):
            if c + 1 < n_chunks:
                copies[c + 1] = start_copy(c + 1)
            copies[c].wait()
            acc = acc + jnp.sum(bufs[c % 2][...], axis=0, keepdims=True)
        partial = acc.astype(x_ref.dtype)
        comm_ref[my, :, :] = partial

        pl.semaphore_wait(barrier, N_DEV - 1)

        sends = []
        for k in range(1, N_DEV):
            rdma = pltpu.make_async_remote_copy(
                src_ref=comm_ref.at[my],
                dst_ref=comm_ref.at[my],
                send_sem=send_sems.at[k - 1],
                recv_sem=recv_sems.at[k - 1],
                device_id=((my + k) % N_DEV,),
                device_id_type=pl.DeviceIdType.MESH,
            )
            rdma.start()
            sends.append(rdma)

        acc = partial
        for k in (1, 3, 2):
            src = (my - k) % N_DEV
            recv = pltpu.make_async_remote_copy(
                src_ref=comm_ref.at[my],
                dst_ref=comm_ref.at[src],
                send_sem=send_sems.at[k - 1],
                recv_sem=recv_sems.at[k - 1],
                device_id=((my + k) % N_DEV,),
                device_id_type=pl.DeviceIdType.MESH,
            )
            recv.wait_recv()
            acc = acc + comm_ref[src, :, :]
        out_ref[...] = acc

        for rdma in sends:
            rdma.wait_send()

    return pl.pallas_call(
        body,
        out_shape=jax.ShapeDtypeStruct((1, n), x.dtype),
        in_specs=[pl.BlockSpec(memory_space=pl.ANY)],
        out_specs=pl.BlockSpec(memory_space=pltpu.VMEM),
        scratch_shapes=[
            pltpu.VMEM((N_DEV, 1, n), x.dtype),
            pltpu.VMEM((CM, n), x.dtype),
            pltpu.VMEM((CM, n), x.dtype),
            pltpu.SemaphoreType.DMA((2,)),
            pltpu.SemaphoreType.DMA((N_DEV - 1,)),
            pltpu.SemaphoreType.DMA((N_DEV - 1,)),
        ],
        compiler_params=pltpu.CompilerParams(collective_id=0),
    )(x)


# device time: 12885 ns/iter; 1.0050x vs baseline; 1.0050x over previous
import jax
import jax.numpy as jnp
from jax import lax
from jax.experimental import pallas as pl
from jax.experimental.pallas import tpu as pltpu

N_DEV = 4


def kernel(x):
    m, n = x.shape

    CM = 512
    n_chunks = m // CM

    def body(x_ref, out_ref, comm_ref, buf0_ref, buf1_ref, copy_sems,
             send_sems, recv_sems):
        bufs = (buf0_ref, buf1_ref)
        my = lax.axis_index("i")

        barrier = pltpu.get_barrier_semaphore()
        for k in range(1, N_DEV):
            pl.semaphore_signal(
                barrier,
                inc=1,
                device_id=((my + k) % N_DEV,),
                device_id_type=pl.DeviceIdType.MESH,
            )

        def start_copy(c):
            cp = pltpu.make_async_copy(
                x_ref.at[pl.ds(c * CM, CM), :],
                bufs[c % 2],
                copy_sems.at[c % 2],
            )
            cp.start()
            return cp

        copies = {0: start_copy(0)}
        acc = jnp.zeros((1, n), jnp.float32)
        for c in range(n_chunks):
            if c + 1 < n_chunks:
                copies[c + 1] = start_copy(c + 1)
            copies[c].wait()
            acc = acc + jnp.sum(bufs[c % 2][...], axis=0, keepdims=True)
        partial = acc.astype(x_ref.dtype)
        comm_ref[my, :, :] = partial

        pl.semaphore_wait(barrier, N_DEV - 1)

        sends = []
        for k in range(1, N_DEV):
            rdma = pltpu.make_async_remote_copy(
                src_ref=comm_ref.at[my],
                dst_ref=comm_ref.at[my],
                send_sem=send_sems.at[k - 1],
                recv_sem=recv_sems.at[k - 1],
                device_id=((my + k) % N_DEV,),
                device_id_type=pl.DeviceIdType.MESH,
            )
            rdma.start()
            sends.append(rdma)

        acc = partial
        for k in (1, 3, 2):
            src = (my - k) % N_DEV
            recv = pltpu.make_async_remote_copy(
                src_ref=comm_ref.at[my],
                dst_ref=comm_ref.at[src],
                send_sem=send_sems.at[k - 1],
                recv_sem=recv_sems.at[k - 1],
                device_id=((my + k) % N_DEV,),
                device_id_type=pl.DeviceIdType.MESH,
            )
            recv.wait_recv()
            acc = acc + comm_ref[src, :, :]
        out_ref[...] = acc

        for rdma in sends:
            rdma.wait_send()

    return pl.pallas_call(
        body,
        out_shape=jax.ShapeDtypeStruct((1, n), x.dtype),
        in_specs=[pl.BlockSpec(memory_space=pltpu.MemorySpace.HBM)],
        out_specs=pl.BlockSpec(memory_space=pltpu.VMEM),
        scratch_shapes=[
            pltpu.VMEM((N_DEV, 1, n), x.dtype),
            pltpu.VMEM((CM, n), x.dtype),
            pltpu.VMEM((CM, n), x.dtype),
            pltpu.SemaphoreType.DMA((2,)),
            pltpu.SemaphoreType.DMA((N_DEV - 1,)),
            pltpu.SemaphoreType.DMA((N_DEV - 1,)),
        ],
        compiler_params=pltpu.CompilerParams(collective_id=0),
    )(pltpu.with_memory_space_constraint(x, pltpu.MemorySpace.HBM))


# device time: 11090 ns/iter; 1.1677x vs baseline; 1.1619x over previous
import jax
import jax.numpy as jnp
from jax import lax
from jax.experimental import pallas as pl
from jax.experimental.pallas import tpu as pltpu

N_DEV = 4


def kernel(x):
    m, n = x.shape

    CM = 512
    n_chunks = m // CM

    def body(x_ref, out_ref, comm_ref, xv_ref, copy_sems, send_sems, recv_sems):
        my = lax.axis_index("i")

        barrier = pltpu.get_barrier_semaphore()
        for k in range(1, N_DEV):
            pl.semaphore_signal(
                barrier,
                inc=1,
                device_id=((my + k) % N_DEV,),
                device_id_type=pl.DeviceIdType.MESH,
            )

        copies = []
        for c in range(n_chunks):
            cp = pltpu.make_async_copy(
                x_ref.at[pl.ds(c * CM, CM), :],
                xv_ref.at[pl.ds(c * CM, CM), :],
                copy_sems.at[c],
            )
            cp.start()
            copies.append(cp)

        acc = jnp.zeros((1, n), jnp.float32)
        for c in range(n_chunks):
            copies[c].wait()
            acc = acc + jnp.sum(
                xv_ref[pl.ds(c * CM, CM), :], axis=0, keepdims=True
            )
        partial = acc.astype(x_ref.dtype)
        comm_ref[my, :, :] = partial

        pl.semaphore_wait(barrier, N_DEV - 1)

        sends = []
        for k in range(1, N_DEV):
            rdma = pltpu.make_async_remote_copy(
                src_ref=comm_ref.at[my],
                dst_ref=comm_ref.at[my],
                send_sem=send_sems.at[k - 1],
                recv_sem=recv_sems.at[k - 1],
                device_id=((my + k) % N_DEV,),
                device_id_type=pl.DeviceIdType.MESH,
            )
            rdma.start()
            sends.append(rdma)

        acc = partial
        for k in (1, 3, 2):
            src = (my - k) % N_DEV
            recv = pltpu.make_async_remote_copy(
                src_ref=comm_ref.at[my],
                dst_ref=comm_ref.at[src],
                send_sem=send_sems.at[k - 1],
                recv_sem=recv_sems.at[k - 1],
                device_id=((my + k) % N_DEV,),
                device_id_type=pl.DeviceIdType.MESH,
            )
            recv.wait_recv()
            acc = acc + comm_ref[src, :, :]
        out_ref[...] = acc

        for rdma in sends:
            rdma.wait_send()

    return pl.pallas_call(
        body,
        out_shape=jax.ShapeDtypeStruct((1, n), x.dtype),
        in_specs=[pl.BlockSpec(memory_space=pltpu.MemorySpace.HBM)],
        out_specs=pl.BlockSpec(memory_space=pltpu.VMEM),
        scratch_shapes=[
            pltpu.VMEM((N_DEV, 1, n), x.dtype),
            pltpu.VMEM((m, n), x.dtype),
            pltpu.SemaphoreType.DMA((m // CM,)),
            pltpu.SemaphoreType.DMA((N_DEV - 1,)),
            pltpu.SemaphoreType.DMA((N_DEV - 1,)),
        ],
        compiler_params=pltpu.CompilerParams(collective_id=0),
    )(pltpu.with_memory_space_constraint(x, pltpu.MemorySpace.HBM))
